# Initial kernel scaffold; baseline (speedup 1.0000x reference)
import jax
import jax.numpy as jnp
from jax.experimental import pallas as pl


def kernel(x, edge_index, W1, W2, W3):
    raise NotImplementedError("write your pallas kernel here")


# speedup vs baseline: 11.0576x; 11.0576x over previous
import functools

import jax
import jax.numpy as jnp
from jax import lax
from jax.experimental import pallas as pl
from jax.experimental.pallas import tpu as pltpu
from jax.experimental.pallas import tpu_sc as plsc

NC = 2
NS = 16
CH = 80
FW = 64
RS = 624
PAD = 8
NQ = 4


def _striped_rows(s, n, fn):
  r0 = pl.multiple_of(s * RS, 8)
  last = n - (NS - 1) * RS

  @pl.when(s < NS - 1)
  def _():
    fn(r0, RS)

  @pl.when(s == NS - 1)
  def _():
    fn(r0, last)


def _agg_body(split_feat, N, rpt, table_h, src_h, dst_h, zero_h, out_h,
              table_sp, acc_sp, idx_s, idx_d, r_a, r_b, sg_a, sg_b, ss_a, ss_b):
  c = lax.axis_index("c")
  s = lax.axis_index("s")

  if split_feat:
    @pl.when(c == 0)
    def _():
      _striped_rows(s, N, lambda r0, nr: pltpu.sync_copy(
          table_h.at[0, pl.ds(r0, nr), :], table_sp.at[pl.ds(r0, nr), :]))

    @pl.when(c == 1)
    def _():
      _striped_rows(s, N, lambda r0, nr: pltpu.sync_copy(
          table_h.at[1, pl.ds(r0, nr), :], table_sp.at[pl.ds(r0, nr), :]))
  else:
    _striped_rows(s, N, lambda r0, nr: pltpu.sync_copy(
        table_h.at[pl.ds(r0, nr), :], table_sp.at[pl.ds(r0, nr), :]))
  _striped_rows(s, N, lambda r0, nr: pltpu.sync_copy(
      zero_h.at[pl.ds(r0, nr), :], acc_sp.at[pl.ds(r0, nr), :]))

  plsc.subcore_barrier()

  w = s if split_feat else c * NS + s
  qr = rpt // NQ

  def pair(jj):
    j0 = 2 * jj
    j1 = j0 + 1
    g0 = pltpu.async_copy(table_sp.at[idx_s.at[j0]], r_a, sg_a)
    g1 = pltpu.async_copy(table_sp.at[idx_s.at[j1]], r_b, sg_b)
    g0.wait()
    s0 = pltpu.async_copy(r_a, acc_sp.at[idx_d.at[j0]], ss_a, add=True)
    g1.wait()
    s1 = pltpu.async_copy(r_b, acc_sp.at[idx_d.at[j1]], ss_b, add=True)
    s0.wait()
    s1.wait()

  def quarter(q):
    q0 = pl.multiple_of(q * qr, 8)
    pltpu.sync_copy(src_h.at[w, pl.ds(q0, qr), :], idx_s)
    pltpu.sync_copy(dst_h.at[w, pl.ds(q0, qr), :], idx_d)
    lax.fori_loop(0, qr // 2, lambda jj, _: (pair(jj), 0)[1], 0)

  lax.fori_loop(0, NQ, lambda q, _: (quarter(q), 0)[1], 0)

  plsc.subcore_barrier()

  @pl.when(c == 0)
  def _():
    _striped_rows(s, N, lambda r0, nr: pltpu.sync_copy(
        acc_sp.at[pl.ds(r0, nr), :], out_h.at[0, pl.ds(r0, nr), :]))

  @pl.when(c == 1)
  def _():
    _striped_rows(s, N, lambda r0, nr: pltpu.sync_copy(
        acc_sp.at[pl.ds(r0, nr), :], out_h.at[1, pl.ds(r0, nr), :]))


def _round_up(a, b):
  return (a + b - 1) // b * b


def _make_agg(N, E, split_feat):
  nrows = E // CH
  rpt = nrows // NS if split_feat else nrows // (NS * NC)
  rpt = _round_up(rpt, NQ * 8)
  mesh = plsc.VectorSubcoreMesh(core_axis_name="c", subcore_axis_name="s")
  return pl.kernel(
      functools.partial(_agg_body, split_feat, N, rpt),
      out_type=jax.ShapeDtypeStruct((2, N, FW), jnp.float32),
      mesh=mesh,
      scratch_types=[
          pltpu.VMEM_SHARED((N + PAD, FW), jnp.float32),
          pltpu.VMEM_SHARED((N + PAD, FW), jnp.float32),
          pltpu.VMEM((rpt // NQ, CH), jnp.int32),
          pltpu.VMEM((rpt // NQ, CH), jnp.int32),
          pltpu.VMEM((CH, FW), jnp.float32),
          pltpu.VMEM((CH, FW), jnp.float32),
          pltpu.SemaphoreType.DMA,
          pltpu.SemaphoreType.DMA,
          pltpu.SemaphoreType.DMA,
          pltpu.SemaphoreType.DMA,
      ],
      name="gcn_agg",
      compiler_params=pltpu.CompilerParams(use_tc_tiling_on_sc=False),
  )


def _deg_body(N, rpt, dst_h, ones_h, zero_h, out_h, acc_sp, idx_d, ones_v, sem):
  c = lax.axis_index("c")
  s = lax.axis_index("s")
  _striped_rows(s, N, lambda r0, nr: pltpu.sync_copy(
      zero_h.at[pl.ds(r0, nr), :], acc_sp.at[pl.ds(r0, nr), :]))
  pltpu.sync_copy(ones_h, ones_v)
  pltpu.sync_copy(dst_h.at[c * NS + s], idx_d)
  plsc.subcore_barrier()

  def chunk(j):
    sc = pltpu.async_copy(ones_v, acc_sp.at[idx_d.at[j]], sem, add=True)
    sc.wait()

  lax.fori_loop(0, rpt, lambda j, _: (chunk(j), 0)[1], 0)
  plsc.subcore_barrier()

  @pl.when(c == 0)
  def _():
    _striped_rows(s, N, lambda r0, nr: pltpu.sync_copy(
        acc_sp.at[pl.ds(r0, nr), :], out_h.at[0, pl.ds(r0, nr), :]))

  @pl.when(c == 1)
  def _():
    _striped_rows(s, N, lambda r0, nr: pltpu.sync_copy(
        acc_sp.at[pl.ds(r0, nr), :], out_h.at[1, pl.ds(r0, nr), :]))


def _make_deg(N, E):
  rpt = _round_up(E // CH // (NS * NC), NQ * 8)
  mesh = plsc.VectorSubcoreMesh(core_axis_name="c", subcore_axis_name="s")
  return pl.kernel(
      functools.partial(_deg_body, N, rpt),
      out_type=jax.ShapeDtypeStruct((2, N, 16), jnp.float32),
      mesh=mesh,
      scratch_types=[
          pltpu.VMEM_SHARED((N + PAD, 16), jnp.float32),
          pltpu.VMEM((rpt, CH), jnp.int32),
          pltpu.VMEM((CH, 16), jnp.float32),
          pltpu.SemaphoreType.DMA,
      ],
      name="gcn_deg",
      compiler_params=pltpu.CompilerParams(use_tc_tiling_on_sc=False),
  )


def _dinv_of(degp_ref):
  deg = degp_ref[0, :, 0] + degp_ref[1, :, 0]
  return lax.rsqrt(jnp.maximum(deg, 1.0))


def _tc1_body(x_ref, w_ref, degp_ref, p_ref):
  dinv = _dinv_of(degp_ref)
  y = jnp.dot(x_ref[...], w_ref[...], preferred_element_type=jnp.float32)
  y = y * dinv[:, None]
  p_ref[0] = y[:, :FW]
  p_ref[1] = y[:, FW:]


def _tcmid_body(split_out, sh_ref, degp_ref, w_ref, p_ref):
  dinv = _dinv_of(degp_ref)
  h = jnp.concatenate([sh_ref[0], sh_ref[1]], axis=1)
  h = jnp.maximum(h * dinv[:, None], 0.0)
  y = jnp.dot(h, w_ref[...], preferred_element_type=jnp.float32)
  y = y * dinv[:, None]
  if split_out:
    p_ref[0] = y[:, :FW]
    p_ref[1] = y[:, FW:]
  else:
    p_ref[...] = y


def _tc4_body(ncls, sp_ref, degp_ref, out_ref):
  dinv = _dinv_of(degp_ref)
  sv = (sp_ref[0] + sp_ref[1]) * dinv[:, None]
  col = lax.broadcasted_iota(jnp.int32, sv.shape, 1)
  sv = jnp.where(col < ncls, sv, -1e30)
  m = jnp.max(sv, axis=1, keepdims=True)
  lse = jnp.log(jnp.sum(jnp.exp(sv - m), axis=1, keepdims=True)) + m
  out_ref[...] = sv - lse


def kernel(x, edge_index, W1, W2, W3):
  N, F = x.shape
  E = edge_index.shape[1]
  H = W1.shape[1]
  C = W3.shape[1]
  f32 = jnp.float32

  ei = edge_index.astype(jnp.int32)

  def _tile_idx(v, ntiles):
    m = v.reshape(ntiles, -1, CH)
    pad_rows = _round_up(m.shape[1], NQ * 8) - m.shape[1]
    padv = jnp.full((ntiles, pad_rows, CH), N, jnp.int32)
    return jnp.concatenate([m, padv], axis=1)

  src16 = _tile_idx(ei[0], NS)
  dst16 = _tile_idx(ei[1], NS)
  src32 = _tile_idx(ei[0], NC * NS)
  dst32 = _tile_idx(ei[1], NC * NS)
  z64 = jnp.zeros((N, FW), f32)
  z16 = jnp.zeros((N, 16), f32)
  ones16 = jnp.ones((CH, 16), f32)
  W3p = jnp.pad(W3, ((0, 0), (0, FW - C)))

  BN = 2000
  grid = (N // BN,)

  degp = _make_deg(N, E)(dst32, ones16, z16)

  tc1 = pl.pallas_call(
      _tc1_body,
      grid=grid,
      in_specs=[
          pl.BlockSpec((BN, F), lambda i: (i, 0)),
          pl.BlockSpec((F, H), lambda i: (0, 0)),
          pl.BlockSpec((2, BN, 16), lambda i: (0, i, 0)),
      ],
      out_specs=pl.BlockSpec((2, BN, FW), lambda i: (0, i, 0)),
      out_shape=jax.ShapeDtypeStruct((2, N, FW), f32),
  )
  p1 = tc1(x, W1, degp)

  agg_split = _make_agg(N, E, True)
  s1 = agg_split(p1, src16, dst16, z64)

  tc2 = pl.pallas_call(
      functools.partial(_tcmid_body, True),
      grid=grid,
      in_specs=[
          pl.BlockSpec((2, BN, FW), lambda i: (0, i, 0)),
          pl.BlockSpec((2, BN, 16), lambda i: (0, i, 0)),
          pl.BlockSpec((H, H), lambda i: (0, 0)),
      ],
      out_specs=pl.BlockSpec((2, BN, FW), lambda i: (0, i, 0)),
      out_shape=jax.ShapeDtypeStruct((2, N, FW), f32),
  )
  p2 = tc2(s1, degp, W2)

  s2 = agg_split(p2, src16, dst16, z64)

  tc3 = pl.pallas_call(
      functools.partial(_tcmid_body, False),
      grid=grid,
      in_specs=[
          pl.BlockSpec((2, BN, FW), lambda i: (0, i, 0)),
          pl.BlockSpec((2, BN, 16), lambda i: (0, i, 0)),
          pl.BlockSpec((H, FW), lambda i: (0, 0)),
      ],
      out_specs=pl.BlockSpec((BN, FW), lambda i: (i, 0)),
      out_shape=jax.ShapeDtypeStruct((N, FW), f32),
  )
  p3 = tc3(s2, degp, W3p)

  s3 = _make_agg(N, E, False)(p3, src32, dst32, z64)

  tc4 = pl.pallas_call(
      functools.partial(_tc4_body, C),
      grid=grid,
      in_specs=[
          pl.BlockSpec((2, BN, FW), lambda i: (0, i, 0)),
          pl.BlockSpec((2, BN, 16), lambda i: (0, i, 0)),
      ],
      out_specs=pl.BlockSpec((BN, FW), lambda i: (i, 0)),
      out_shape=jax.ShapeDtypeStruct((N, FW), f32),
  )
  out = tc4(s3, degp)
  return out[:, :C]


# speedup vs baseline: 15.2371x; 1.3780x over previous
import functools

import jax
import jax.numpy as jnp
from jax import lax
from jax.experimental import pallas as pl
from jax.experimental.pallas import tpu as pltpu
from jax.experimental.pallas import tpu_sc as plsc

NC = 2
NS = 16
CH = 128
FW = 64
RS = 624
PAD = 8
QR = 40


def _striped_rows(s, n, fn):
  r0 = pl.multiple_of(s * RS, 8)
  last = n - (NS - 1) * RS

  @pl.when(s < NS - 1)
  def _():
    fn(r0, RS)

  @pl.when(s == NS - 1)
  def _():
    fn(r0, last)


def _agg_body(split_feat, N, rpt, table_h, src_h, dst_h, zero_h, out_h,
              table_sp, acc_sp, idx_s, idx_d, r_a0, r_a1, r_b0, r_b1,
              sg_a0, sg_a1, sg_b0, sg_b1, ss_a0, ss_a1, ss_b0, ss_b1):
  c = lax.axis_index("c")
  s = lax.axis_index("s")

  if split_feat:
    @pl.when(c == 0)
    def _():
      _striped_rows(s, N, lambda r0, nr: pltpu.sync_copy(
          table_h.at[0, pl.ds(r0, nr), :], table_sp.at[pl.ds(r0, nr), :]))

    @pl.when(c == 1)
    def _():
      _striped_rows(s, N, lambda r0, nr: pltpu.sync_copy(
          table_h.at[1, pl.ds(r0, nr), :], table_sp.at[pl.ds(r0, nr), :]))
  else:
    _striped_rows(s, N, lambda r0, nr: pltpu.sync_copy(
        table_h.at[pl.ds(r0, nr), :], table_sp.at[pl.ds(r0, nr), :]))
  _striped_rows(s, N, lambda r0, nr: pltpu.sync_copy(
      zero_h.at[pl.ds(r0, nr), :], acc_sp.at[pl.ds(r0, nr), :]))

  plsc.subcore_barrier()

  w = s if split_feat else c * NS + s
  nq = rpt // QR
  nsup = QR // 4

  def start_g(j, buf, sem):
    pltpu.async_copy(table_sp.at[idx_s.at[j]], buf, sem)

  def wait_g(buf, sem):
    pltpu.make_async_copy(table_sp.at[idx_s.at[0]], buf, sem).wait()

  def start_s(j, buf, sem):
    pltpu.async_copy(buf, acc_sp.at[idx_d.at[j]], sem, add=True)

  def wait_s(buf, sem):
    pltpu.make_async_copy(buf, acc_sp.at[idx_d.at[0]], sem).wait()

  def superiter(u):
    j = 4 * u
    wait_g(r_a0, sg_a0)
    wait_g(r_a1, sg_a1)

    @pl.when(u > 0)
    def _():
      wait_s(r_b0, ss_b0)
      wait_s(r_b1, ss_b1)

    start_g(j + 2, r_b0, sg_b0)
    start_g(j + 3, r_b1, sg_b1)
    start_s(j, r_a0, ss_a0)
    start_s(j + 1, r_a1, ss_a1)
    wait_g(r_b0, sg_b0)
    wait_g(r_b1, sg_b1)
    wait_s(r_a0, ss_a0)
    wait_s(r_a1, ss_a1)

    @pl.when(u < nsup - 1)
    def _():
      start_g(j + 4, r_a0, sg_a0)
      start_g(j + 5, r_a1, sg_a1)
    start_s(j + 2, r_b0, ss_b0)
    start_s(j + 3, r_b1, ss_b1)

  def quarter(q):
    q0 = pl.multiple_of(q * QR, 8)
    pltpu.sync_copy(src_h.at[w, pl.ds(q0, QR), :], idx_s)
    pltpu.sync_copy(dst_h.at[w, pl.ds(q0, QR), :], idx_d)
    start_g(0, r_a0, sg_a0)
    start_g(1, r_a1, sg_a1)
    lax.fori_loop(0, nsup, lambda u, _: (superiter(u), 0)[1], 0)
    wait_s(r_b0, ss_b0)
    wait_s(r_b1, ss_b1)

  lax.fori_loop(0, nq, lambda q, _: (quarter(q), 0)[1], 0)

  plsc.subcore_barrier()

  @pl.when(c == 0)
  def _():
    _striped_rows(s, N, lambda r0, nr: pltpu.sync_copy(
        acc_sp.at[pl.ds(r0, nr), :], out_h.at[0, pl.ds(r0, nr), :]))

  @pl.when(c == 1)
  def _():
    _striped_rows(s, N, lambda r0, nr: pltpu.sync_copy(
        acc_sp.at[pl.ds(r0, nr), :], out_h.at[1, pl.ds(r0, nr), :]))


def _round_up(a, b):
  return (a + b - 1) // b * b


def _tile_rows(E, ntiles):
  return _round_up(-(-E // (ntiles * CH)), QR)


def _make_agg(N, E, split_feat):
  rpt = _tile_rows(E, NS if split_feat else NS * NC)
  mesh = plsc.VectorSubcoreMesh(core_axis_name="c", subcore_axis_name="s")
  return pl.kernel(
      functools.partial(_agg_body, split_feat, N, rpt),
      out_type=jax.ShapeDtypeStruct((2, N, FW), jnp.float32),
      mesh=mesh,
      scratch_types=[
          pltpu.VMEM_SHARED((N + PAD, FW), jnp.float32),
          pltpu.VMEM_SHARED((N + PAD, FW), jnp.float32),
          pltpu.VMEM((QR, CH), jnp.int32),
          pltpu.VMEM((QR, CH), jnp.int32),
          pltpu.VMEM((CH, FW), jnp.float32),
          pltpu.VMEM((CH, FW), jnp.float32),
          pltpu.VMEM((CH, FW), jnp.float32),
          pltpu.VMEM((CH, FW), jnp.float32),
          pltpu.SemaphoreType.DMA,
          pltpu.SemaphoreType.DMA,
          pltpu.SemaphoreType.DMA,
          pltpu.SemaphoreType.DMA,
          pltpu.SemaphoreType.DMA,
          pltpu.SemaphoreType.DMA,
          pltpu.SemaphoreType.DMA,
          pltpu.SemaphoreType.DMA,
      ],
      name="gcn_agg",
      compiler_params=pltpu.CompilerParams(use_tc_tiling_on_sc=False),
  )


def _deg_body(N, rpt, dst_h, ones_h, zero_h, out_h, acc_sp, idx_d, ones_v, sem):
  c = lax.axis_index("c")
  s = lax.axis_index("s")
  _striped_rows(s, N, lambda r0, nr: pltpu.sync_copy(
      zero_h.at[pl.ds(r0, nr), :], acc_sp.at[pl.ds(r0, nr), :]))
  pltpu.sync_copy(ones_h, ones_v)
  pltpu.sync_copy(dst_h.at[c * NS + s], idx_d)
  plsc.subcore_barrier()

  def burst(b):
    lax.fori_loop(0, 20, lambda j, _: (
        pltpu.async_copy(ones_v, acc_sp.at[idx_d.at[b * 20 + j]], sem,
                         add=True), 0)[1], 0)
    lax.fori_loop(0, 20, lambda j, _: (
        pltpu.make_async_copy(ones_v, acc_sp.at[idx_d.at[0]], sem).wait(),
        0)[1], 0)

  lax.fori_loop(0, rpt // 20, lambda b, _: (burst(b), 0)[1], 0)
  plsc.subcore_barrier()

  @pl.when(c == 0)
  def _():
    _striped_rows(s, N, lambda r0, nr: pltpu.sync_copy(
        acc_sp.at[pl.ds(r0, nr), :], out_h.at[0, pl.ds(r0, nr), :]))

  @pl.when(c == 1)
  def _():
    _striped_rows(s, N, lambda r0, nr: pltpu.sync_copy(
        acc_sp.at[pl.ds(r0, nr), :], out_h.at[1, pl.ds(r0, nr), :]))


def _make_deg(N, E):
  rpt = _tile_rows(E, NS * NC)
  mesh = plsc.VectorSubcoreMesh(core_axis_name="c", subcore_axis_name="s")
  return pl.kernel(
      functools.partial(_deg_body, N, rpt),
      out_type=jax.ShapeDtypeStruct((2, N, 16), jnp.float32),
      mesh=mesh,
      scratch_types=[
          pltpu.VMEM_SHARED((N + PAD, 16), jnp.float32),
          pltpu.VMEM((rpt, CH), jnp.int32),
          pltpu.VMEM((CH, 16), jnp.float32),
          pltpu.SemaphoreType.DMA,
      ],
      name="gcn_deg",
      compiler_params=pltpu.CompilerParams(use_tc_tiling_on_sc=False),
  )


def _dinv_of(degp_ref):
  deg = degp_ref[0, :, 0] + degp_ref[1, :, 0]
  return lax.rsqrt(jnp.maximum(deg, 1.0))


def _tc1_body(x_ref, w_ref, degp_ref, p_ref):
  dinv = _dinv_of(degp_ref)
  y = jnp.dot(x_ref[...], w_ref[...], preferred_element_type=jnp.float32)
  y = y * dinv[:, None]
  p_ref[0] = y[:, :FW]
  p_ref[1] = y[:, FW:]


def _tcmid_body(split_out, sh_ref, degp_ref, w_ref, p_ref):
  dinv = _dinv_of(degp_ref)
  h = jnp.concatenate([sh_ref[0], sh_ref[1]], axis=1)
  h = jnp.maximum(h * dinv[:, None], 0.0)
  y = jnp.dot(h, w_ref[...], preferred_element_type=jnp.float32)
  y = y * dinv[:, None]
  if split_out:
    p_ref[0] = y[:, :FW]
    p_ref[1] = y[:, FW:]
  else:
    p_ref[...] = y


def _tc4_body(ncls, sp_ref, degp_ref, out_ref):
  dinv = _dinv_of(degp_ref)
  sv = (sp_ref[0] + sp_ref[1]) * dinv[:, None]
  col = lax.broadcasted_iota(jnp.int32, sv.shape, 1)
  sv = jnp.where(col < ncls, sv, -1e30)
  m = jnp.max(sv, axis=1, keepdims=True)
  lse = jnp.log(jnp.sum(jnp.exp(sv - m), axis=1, keepdims=True)) + m
  out_ref[...] = sv - lse


def kernel(x, edge_index, W1, W2, W3):
  N, F = x.shape
  E = edge_index.shape[1]
  H = W1.shape[1]
  C = W3.shape[1]
  f32 = jnp.float32

  ei = edge_index.astype(jnp.int32)

  def _tile_idx(v, ntiles):
    m = v.reshape(ntiles, -1)
    pad = _tile_rows(E, ntiles) * CH - m.shape[1]
    padv = jnp.full((ntiles, pad), N, jnp.int32)
    return jnp.concatenate([m, padv], axis=1).reshape(ntiles, -1, CH)

  src16 = _tile_idx(ei[0], NS)
  dst16 = _tile_idx(ei[1], NS)
  src32 = _tile_idx(ei[0], NC * NS)
  dst32 = _tile_idx(ei[1], NC * NS)
  z64 = jnp.zeros((N, FW), f32)
  z16 = jnp.zeros((N, 16), f32)
  ones16 = jnp.ones((CH, 16), f32)
  W3p = jnp.pad(W3, ((0, 0), (0, FW - C)))

  BN = 2000
  grid = (N // BN,)

  degp = _make_deg(N, E)(dst32, ones16, z16)

  tc1 = pl.pallas_call(
      _tc1_body,
      grid=grid,
      in_specs=[
          pl.BlockSpec((BN, F), lambda i: (i, 0)),
          pl.BlockSpec((F, H), lambda i: (0, 0)),
          pl.BlockSpec((2, BN, 16), lambda i: (0, i, 0)),
      ],
      out_specs=pl.BlockSpec((2, BN, FW), lambda i: (0, i, 0)),
      out_shape=jax.ShapeDtypeStruct((2, N, FW), f32),
  )
  p1 = tc1(x, W1, degp)

  agg_split = _make_agg(N, E, True)
  s1 = agg_split(p1, src16, dst16, z64)

  tc2 = pl.pallas_call(
      functools.partial(_tcmid_body, True),
      grid=grid,
      in_specs=[
          pl.BlockSpec((2, BN, FW), lambda i: (0, i, 0)),
          pl.BlockSpec((2, BN, 16), lambda i: (0, i, 0)),
          pl.BlockSpec((H, H), lambda i: (0, 0)),
      ],
      out_specs=pl.BlockSpec((2, BN, FW), lambda i: (0, i, 0)),
      out_shape=jax.ShapeDtypeStruct((2, N, FW), f32),
  )
  p2 = tc2(s1, degp, W2)

  s2 = agg_split(p2, src16, dst16, z64)

  tc3 = pl.pallas_call(
      functools.partial(_tcmid_body, False),
      grid=grid,
      in_specs=[
          pl.BlockSpec((2, BN, FW), lambda i: (0, i, 0)),
          pl.BlockSpec((2, BN, 16), lambda i: (0, i, 0)),
          pl.BlockSpec((H, FW), lambda i: (0, 0)),
      ],
      out_specs=pl.BlockSpec((BN, FW), lambda i: (i, 0)),
      out_shape=jax.ShapeDtypeStruct((N, FW), f32),
  )
  p3 = tc3(s2, degp, W3p)

  s3 = _make_agg(N, E, False)(p3, src32, dst32, z64)

  tc4 = pl.pallas_call(
      functools.partial(_tc4_body, C),
      grid=grid,
      in_specs=[
          pl.BlockSpec((2, BN, FW), lambda i: (0, i, 0)),
          pl.BlockSpec((2, BN, 16), lambda i: (0, i, 0)),
      ],
      out_specs=pl.BlockSpec((BN, FW), lambda i: (i, 0)),
      out_shape=jax.ShapeDtypeStruct((N, FW), f32),
  )
  out = tc4(s3, degp)
  return out[:, :C]


# speedup vs baseline: 15.8183x; 1.0381x over previous
import functools

import jax
import jax.numpy as jnp
from jax import lax
from jax.experimental import pallas as pl
from jax.experimental.pallas import tpu as pltpu
from jax.experimental.pallas import tpu_sc as plsc

NC = 2
NS = 16
CH = 128
FW = 64
RS = 624
PAD = 8
QR = 40


def _striped_rows(s, n, fn):
  r0 = pl.multiple_of(s * RS, 8)
  last = n - (NS - 1) * RS

  @pl.when(s < NS - 1)
  def _():
    fn(r0, RS)

  @pl.when(s == NS - 1)
  def _():
    fn(r0, last)


def _agg_body(split_feat, N, rpt, table_h, src_h, dst_h, zero_h, out_h,
              acc_sp, idx_s, idx_d, r_a0, r_a1, r_b0, r_b1,
              sg_a0, sg_a1, sg_b0, sg_b1, ss_a0, ss_a1, ss_b0, ss_b1):
  c = lax.axis_index("c")
  s = lax.axis_index("s")

  _striped_rows(s, N, lambda r0, nr: pltpu.sync_copy(
      zero_h.at[pl.ds(r0, nr), :], acc_sp.at[pl.ds(r0, nr), :]))

  plsc.subcore_barrier()

  w = s if split_feat else c * NS + s
  nq = rpt // QR
  nsup = QR // 4

  def pipeline(tbl):
    def start_g(j, buf, sem):
      pltpu.async_copy(tbl.at[idx_s.at[j]], buf, sem)

    def wait_g(buf, sem):
      pltpu.make_async_copy(tbl.at[idx_s.at[0]], buf, sem).wait()

    def start_s(j, buf, sem):
      pltpu.async_copy(buf, acc_sp.at[idx_d.at[j]], sem, add=True)

    def wait_s(buf, sem):
      pltpu.make_async_copy(buf, acc_sp.at[idx_d.at[0]], sem).wait()

    def superiter(u):
      j = 4 * u
      wait_g(r_a0, sg_a0)
      wait_g(r_a1, sg_a1)

      @pl.when(u > 0)
      def _():
        wait_s(r_b0, ss_b0)
        wait_s(r_b1, ss_b1)

      start_g(j + 2, r_b0, sg_b0)
      start_g(j + 3, r_b1, sg_b1)
      start_s(j, r_a0, ss_a0)
      start_s(j + 1, r_a1, ss_a1)
      wait_g(r_b0, sg_b0)
      wait_g(r_b1, sg_b1)
      wait_s(r_a0, ss_a0)
      wait_s(r_a1, ss_a1)

      @pl.when(u < nsup - 1)
      def _():
        start_g(j + 4, r_a0, sg_a0)
        start_g(j + 5, r_a1, sg_a1)
      start_s(j + 2, r_b0, ss_b0)
      start_s(j + 3, r_b1, ss_b1)

    def quarter(q):
      q0 = pl.multiple_of(q * QR, 8)
      pltpu.sync_copy(src_h.at[w, pl.ds(q0, QR), :], idx_s)
      pltpu.sync_copy(dst_h.at[w, pl.ds(q0, QR), :], idx_d)
      start_g(0, r_a0, sg_a0)
      start_g(1, r_a1, sg_a1)
      lax.fori_loop(0, nsup, lambda u, _: (superiter(u), 0)[1], 0)
      wait_s(r_b0, ss_b0)
      wait_s(r_b1, ss_b1)

    lax.fori_loop(0, nq, lambda q, _: (quarter(q), 0)[1], 0)

  if split_feat:
    @pl.when(c == 0)
    def _():
      pipeline(table_h.at[0])

    @pl.when(c == 1)
    def _():
      pipeline(table_h.at[1])
  else:
    pipeline(table_h)

  plsc.subcore_barrier()

  @pl.when(c == 0)
  def _():
    _striped_rows(s, N, lambda r0, nr: pltpu.sync_copy(
        acc_sp.at[pl.ds(r0, nr), :], out_h.at[0, pl.ds(r0, nr), :]))

  @pl.when(c == 1)
  def _():
    _striped_rows(s, N, lambda r0, nr: pltpu.sync_copy(
        acc_sp.at[pl.ds(r0, nr), :], out_h.at[1, pl.ds(r0, nr), :]))


def _round_up(a, b):
  return (a + b - 1) // b * b


def _tile_rows(E, ntiles):
  return _round_up(-(-E // (ntiles * CH)), QR)


def _make_agg(N, E, split_feat):
  rpt = _tile_rows(E, NS if split_feat else NS * NC)
  mesh = plsc.VectorSubcoreMesh(core_axis_name="c", subcore_axis_name="s")
  return pl.kernel(
      functools.partial(_agg_body, split_feat, N, rpt),
      out_type=jax.ShapeDtypeStruct((2, N, FW), jnp.float32),
      mesh=mesh,
      scratch_types=[
          pltpu.VMEM_SHARED((N + PAD, FW), jnp.float32),
          pltpu.VMEM((QR, CH), jnp.int32),
          pltpu.VMEM((QR, CH), jnp.int32),
          pltpu.VMEM((CH, FW), jnp.float32),
          pltpu.VMEM((CH, FW), jnp.float32),
          pltpu.VMEM((CH, FW), jnp.float32),
          pltpu.VMEM((CH, FW), jnp.float32),
          pltpu.SemaphoreType.DMA,
          pltpu.SemaphoreType.DMA,
          pltpu.SemaphoreType.DMA,
          pltpu.SemaphoreType.DMA,
          pltpu.SemaphoreType.DMA,
          pltpu.SemaphoreType.DMA,
          pltpu.SemaphoreType.DMA,
          pltpu.SemaphoreType.DMA,
      ],
      name="gcn_agg",
      compiler_params=pltpu.CompilerParams(use_tc_tiling_on_sc=False),
  )


def _deg_body(N, rpt, dst_h, ones_h, zero_h, out_h, acc_sp, idx_d, ones_v, sem):
  c = lax.axis_index("c")
  s = lax.axis_index("s")
  _striped_rows(s, N, lambda r0, nr: pltpu.sync_copy(
      zero_h.at[pl.ds(r0, nr), :], acc_sp.at[pl.ds(r0, nr), :]))
  pltpu.sync_copy(ones_h, ones_v)
  pltpu.sync_copy(dst_h.at[c * NS + s], idx_d)
  plsc.subcore_barrier()

  def burst(b):
    lax.fori_loop(0, 20, lambda j, _: (
        pltpu.async_copy(ones_v, acc_sp.at[idx_d.at[b * 20 + j]], sem,
                         add=True), 0)[1], 0)
    lax.fori_loop(0, 20, lambda j, _: (
        pltpu.make_async_copy(ones_v, acc_sp.at[idx_d.at[0]], sem).wait(),
        0)[1], 0)

  lax.fori_loop(0, rpt // 20, lambda b, _: (burst(b), 0)[1], 0)
  plsc.subcore_barrier()

  @pl.when(c == 0)
  def _():
    _striped_rows(s, N, lambda r0, nr: pltpu.sync_copy(
        acc_sp.at[pl.ds(r0, nr), :], out_h.at[0, pl.ds(r0, nr), :]))

  @pl.when(c == 1)
  def _():
    _striped_rows(s, N, lambda r0, nr: pltpu.sync_copy(
        acc_sp.at[pl.ds(r0, nr), :], out_h.at[1, pl.ds(r0, nr), :]))


def _make_deg(N, E):
  rpt = _tile_rows(E, NS * NC)
  mesh = plsc.VectorSubcoreMesh(core_axis_name="c", subcore_axis_name="s")
  return pl.kernel(
      functools.partial(_deg_body, N, rpt),
      out_type=jax.ShapeDtypeStruct((2, N, 16), jnp.float32),
      mesh=mesh,
      scratch_types=[
          pltpu.VMEM_SHARED((N + PAD, 16), jnp.float32),
          pltpu.VMEM((rpt, CH), jnp.int32),
          pltpu.VMEM((CH, 16), jnp.float32),
          pltpu.SemaphoreType.DMA,
      ],
      name="gcn_deg",
      compiler_params=pltpu.CompilerParams(use_tc_tiling_on_sc=False),
  )


def _dinv_of(degp_ref):
  deg = degp_ref[0, :, 0] + degp_ref[1, :, 0]
  return lax.rsqrt(jnp.maximum(deg, 1.0))


def _tc1_body(x_ref, w_ref, degp_ref, p_ref):
  dinv = _dinv_of(degp_ref)
  y = jnp.dot(x_ref[...], w_ref[...], preferred_element_type=jnp.float32)
  y = y * dinv[:, None]
  p_ref[0] = y[:, :FW]
  p_ref[1] = y[:, FW:]


def _tcmid_body(split_out, sh_ref, degp_ref, w_ref, p_ref):
  dinv = _dinv_of(degp_ref)
  h = jnp.concatenate([sh_ref[0], sh_ref[1]], axis=1)
  h = jnp.maximum(h * dinv[:, None], 0.0)
  y = jnp.dot(h, w_ref[...], preferred_element_type=jnp.float32)
  y = y * dinv[:, None]
  if split_out:
    p_ref[0] = y[:, :FW]
    p_ref[1] = y[:, FW:]
  else:
    p_ref[...] = y


def _tc4_body(ncls, sp_ref, degp_ref, out_ref):
  dinv = _dinv_of(degp_ref)
  sv = (sp_ref[0] + sp_ref[1]) * dinv[:, None]
  col = lax.broadcasted_iota(jnp.int32, sv.shape, 1)
  sv = jnp.where(col < ncls, sv, -1e30)
  m = jnp.max(sv, axis=1, keepdims=True)
  lse = jnp.log(jnp.sum(jnp.exp(sv - m), axis=1, keepdims=True)) + m
  out_ref[...] = sv - lse


def kernel(x, edge_index, W1, W2, W3):
  N, F = x.shape
  E = edge_index.shape[1]
  H = W1.shape[1]
  C = W3.shape[1]
  f32 = jnp.float32

  ei = edge_index.astype(jnp.int32)

  def _tile_idx(v, ntiles, is_src):
    m = v.reshape(ntiles, -1)
    pad = _tile_rows(E, ntiles) * CH - m.shape[1]
    if is_src:
      padv = jnp.broadcast_to(
          (jnp.arange(pad, dtype=jnp.int32) * 797) % N, (ntiles, pad))
    else:
      padv = jnp.full((ntiles, pad), N, jnp.int32)
    return jnp.concatenate([m, padv], axis=1).reshape(ntiles, -1, CH)

  src16 = _tile_idx(ei[0], NS, True)
  dst16 = _tile_idx(ei[1], NS, False)
  src32 = _tile_idx(ei[0], NC * NS, True)
  dst32 = _tile_idx(ei[1], NC * NS, False)
  z64 = jnp.zeros((N, FW), f32)
  z16 = jnp.zeros((N, 16), f32)
  ones16 = jnp.ones((CH, 16), f32)
  W3p = jnp.pad(W3, ((0, 0), (0, FW - C)))

  BN = 2000
  grid = (N // BN,)

  degp = _make_deg(N, E)(dst32, ones16, z16)

  tc1 = pl.pallas_call(
      _tc1_body,
      grid=grid,
      in_specs=[
          pl.BlockSpec((BN, F), lambda i: (i, 0)),
          pl.BlockSpec((F, H), lambda i: (0, 0)),
          pl.BlockSpec((2, BN, 16), lambda i: (0, i, 0)),
      ],
      out_specs=pl.BlockSpec((2, BN, FW), lambda i: (0, i, 0)),
      out_shape=jax.ShapeDtypeStruct((2, N, FW), f32),
  )
  p1 = tc1(x, W1, degp)

  agg_split = _make_agg(N, E, True)
  s1 = agg_split(p1, src16, dst16, z64)

  tc2 = pl.pallas_call(
      functools.partial(_tcmid_body, True),
      grid=grid,
      in_specs=[
          pl.BlockSpec((2, BN, FW), lambda i: (0, i, 0)),
          pl.BlockSpec((2, BN, 16), lambda i: (0, i, 0)),
          pl.BlockSpec((H, H), lambda i: (0, 0)),
      ],
      out_specs=pl.BlockSpec((2, BN, FW), lambda i: (0, i, 0)),
      out_shape=jax.ShapeDtypeStruct((2, N, FW), f32),
  )
  p2 = tc2(s1, degp, W2)

  s2 = agg_split(p2, src16, dst16, z64)

  tc3 = pl.pallas_call(
      functools.partial(_tcmid_body, False),
      grid=grid,
      in_specs=[
          pl.BlockSpec((2, BN, FW), lambda i: (0, i, 0)),
          pl.BlockSpec((2, BN, 16), lambda i: (0, i, 0)),
          pl.BlockSpec((H, FW), lambda i: (0, 0)),
      ],
      out_specs=pl.BlockSpec((BN, FW), lambda i: (i, 0)),
      out_shape=jax.ShapeDtypeStruct((N, FW), f32),
  )
  p3 = tc3(s2, degp, W3p)

  s3 = _make_agg(N, E, False)(p3, src32, dst32, z64)

  tc4 = pl.pallas_call(
      functools.partial(_tc4_body, C),
      grid=grid,
      in_specs=[
          pl.BlockSpec((2, BN, FW), lambda i: (0, i, 0)),
          pl.BlockSpec((2, BN, 16), lambda i: (0, i, 0)),
      ],
      out_specs=pl.BlockSpec((BN, FW), lambda i: (i, 0)),
      out_shape=jax.ShapeDtypeStruct((N, FW), f32),
  )
  out = tc4(s3, degp)
  return out[:, :C]


# speedup vs baseline: 16.8715x; 1.0666x over previous
import functools

import jax
import jax.numpy as jnp
from jax import lax
from jax.experimental import pallas as pl
from jax.experimental.pallas import tpu as pltpu
from jax.experimental.pallas import tpu_sc as plsc

NC = 2
NS = 16
CH = 128
FW = 64
RS = 624
PAD = 8
QR = 40


def _striped_rows(s, n, fn):
  r0 = pl.multiple_of(s * RS, 8)
  last = n - (NS - 1) * RS

  @pl.when(s < NS - 1)
  def _():
    fn(r0, RS)

  @pl.when(s == NS - 1)
  def _():
    fn(r0, last)


K = 4


def _agg_body(split_feat, N, rpt, table_h, src_h, dst_h, zero_h, out_h,
              acc_sp, idx_s, idx_d, rbufs, gsems, ssems):
  c = lax.axis_index("c")
  s = lax.axis_index("s")

  _striped_rows(s, N, lambda r0, nr: pltpu.sync_copy(
      zero_h.at[pl.ds(r0, nr), :], acc_sp.at[pl.ds(r0, nr), :]))

  plsc.subcore_barrier()

  w = s if split_feat else c * NS + s
  nq = rpt // QR
  nsup = QR // (2 * K)
  A = range(K)
  B = range(K, 2 * K)

  def pipeline(tbl):
    def start_g(j, b):
      pltpu.async_copy(tbl.at[idx_s.at[j]], rbufs[b], gsems[b])

    def wait_g(b):
      pltpu.make_async_copy(tbl.at[idx_s.at[0]], rbufs[b], gsems[b]).wait()

    def start_s(j, b):
      pltpu.async_copy(rbufs[b], acc_sp.at[idx_d.at[j]], ssems[b], add=True)

    def wait_s(b):
      pltpu.make_async_copy(rbufs[b], acc_sp.at[idx_d.at[0]], ssems[b]).wait()

    def superiter(u):
      j = 2 * K * u
      for k in A:
        wait_g(k)

      @pl.when(u > 0)
      def _():
        for k in B:
          wait_s(k)

      for k in B:
        start_g(j + k, k)
      for k in A:
        start_s(j + k, k)
      for k in B:
        wait_g(k)
      for k in A:
        wait_s(k)

      @pl.when(u < nsup - 1)
      def _():
        for k in A:
          start_g(j + 2 * K + k, k)
      for k in B:
        start_s(j + k, k)

    def quarter(q):
      q0 = pl.multiple_of(q * QR, 8)
      pltpu.sync_copy(src_h.at[w, pl.ds(q0, QR), :], idx_s)
      pltpu.sync_copy(dst_h.at[w, pl.ds(q0, QR), :], idx_d)
      for k in A:
        start_g(k, k)
      lax.fori_loop(0, nsup, lambda u, _: (superiter(u), 0)[1], 0)
      for k in B:
        wait_s(k)

    lax.fori_loop(0, nq, lambda q, _: (quarter(q), 0)[1], 0)

  if split_feat:
    @pl.when(c == 0)
    def _():
      pipeline(table_h.at[0])

    @pl.when(c == 1)
    def _():
      pipeline(table_h.at[1])
  else:
    pipeline(table_h)

  plsc.subcore_barrier()

  @pl.when(c == 0)
  def _():
    _striped_rows(s, N, lambda r0, nr: pltpu.sync_copy(
        acc_sp.at[pl.ds(r0, nr), :], out_h.at[0, pl.ds(r0, nr), :]))

  @pl.when(c == 1)
  def _():
    _striped_rows(s, N, lambda r0, nr: pltpu.sync_copy(
        acc_sp.at[pl.ds(r0, nr), :], out_h.at[1, pl.ds(r0, nr), :]))


def _round_up(a, b):
  return (a + b - 1) // b * b


def _tile_rows(E, ntiles):
  return _round_up(-(-E // (ntiles * CH)), QR)


def _make_agg(N, E, split_feat):
  rpt = _tile_rows(E, NS if split_feat else NS * NC)
  mesh = plsc.VectorSubcoreMesh(core_axis_name="c", subcore_axis_name="s")
  return pl.kernel(
      functools.partial(_agg_body, split_feat, N, rpt),
      out_type=jax.ShapeDtypeStruct((2, N, FW), jnp.float32),
      mesh=mesh,
      scratch_types=[
          pltpu.VMEM_SHARED((N + PAD, FW), jnp.float32),
          pltpu.VMEM((QR, CH), jnp.int32),
          pltpu.VMEM((QR, CH), jnp.int32),
          [pltpu.VMEM((CH, FW), jnp.float32) for _ in range(2 * K)],
          [pltpu.SemaphoreType.DMA for _ in range(2 * K)],
          [pltpu.SemaphoreType.DMA for _ in range(2 * K)],
      ],
      name="gcn_agg",
      compiler_params=pltpu.CompilerParams(use_tc_tiling_on_sc=False),
  )


def _deg_body(N, rpt, dst_h, ones_h, zero_h, out_h, acc_sp, idx_d, ones_v, sem):
  c = lax.axis_index("c")
  s = lax.axis_index("s")
  _striped_rows(s, N, lambda r0, nr: pltpu.sync_copy(
      zero_h.at[pl.ds(r0, nr), :], acc_sp.at[pl.ds(r0, nr), :]))
  pltpu.sync_copy(ones_h, ones_v)
  pltpu.sync_copy(dst_h.at[c * NS + s], idx_d)
  plsc.subcore_barrier()

  def burst(b):
    lax.fori_loop(0, 20, lambda j, _: (
        pltpu.async_copy(ones_v, acc_sp.at[idx_d.at[b * 20 + j]], sem,
                         add=True), 0)[1], 0)
    lax.fori_loop(0, 20, lambda j, _: (
        pltpu.make_async_copy(ones_v, acc_sp.at[idx_d.at[0]], sem).wait(),
        0)[1], 0)

  lax.fori_loop(0, rpt // 20, lambda b, _: (burst(b), 0)[1], 0)
  plsc.subcore_barrier()

  @pl.when(c == 0)
  def _():
    _striped_rows(s, N, lambda r0, nr: pltpu.sync_copy(
        acc_sp.at[pl.ds(r0, nr), :], out_h.at[0, pl.ds(r0, nr), :]))

  @pl.when(c == 1)
  def _():
    _striped_rows(s, N, lambda r0, nr: pltpu.sync_copy(
        acc_sp.at[pl.ds(r0, nr), :], out_h.at[1, pl.ds(r0, nr), :]))


def _make_deg(N, E):
  rpt = _tile_rows(E, NS * NC)
  mesh = plsc.VectorSubcoreMesh(core_axis_name="c", subcore_axis_name="s")
  return pl.kernel(
      functools.partial(_deg_body, N, rpt),
      out_type=jax.ShapeDtypeStruct((2, N, 16), jnp.float32),
      mesh=mesh,
      scratch_types=[
          pltpu.VMEM_SHARED((N + PAD, 16), jnp.float32),
          pltpu.VMEM((rpt, CH), jnp.int32),
          pltpu.VMEM((CH, 16), jnp.float32),
          pltpu.SemaphoreType.DMA,
      ],
      name="gcn_deg",
      compiler_params=pltpu.CompilerParams(use_tc_tiling_on_sc=False),
  )


def _dinv_of(degp_ref):
  deg = degp_ref[0, :, 0] + degp_ref[1, :, 0]
  return lax.rsqrt(jnp.maximum(deg, 1.0))


def _tc1_body(x_ref, w_ref, degp_ref, p_ref):
  dinv = _dinv_of(degp_ref)
  y = jnp.dot(x_ref[...], w_ref[...], preferred_element_type=jnp.float32)
  y = y * dinv[:, None]
  p_ref[0] = y[:, :FW]
  p_ref[1] = y[:, FW:]


def _tcmid_body(split_out, sh_ref, degp_ref, w_ref, p_ref):
  dinv = _dinv_of(degp_ref)
  h = jnp.concatenate([sh_ref[0], sh_ref[1]], axis=1)
  h = jnp.maximum(h * dinv[:, None], 0.0)
  y = jnp.dot(h, w_ref[...], preferred_element_type=jnp.float32)
  y = y * dinv[:, None]
  if split_out:
    p_ref[0] = y[:, :FW]
    p_ref[1] = y[:, FW:]
  else:
    p_ref[...] = y


def _tc4_body(ncls, sp_ref, degp_ref, out_ref):
  dinv = _dinv_of(degp_ref)
  sv = (sp_ref[0] + sp_ref[1]) * dinv[:, None]
  col = lax.broadcasted_iota(jnp.int32, sv.shape, 1)
  sv = jnp.where(col < ncls, sv, -1e30)
  m = jnp.max(sv, axis=1, keepdims=True)
  lse = jnp.log(jnp.sum(jnp.exp(sv - m), axis=1, keepdims=True)) + m
  out_ref[...] = sv - lse


def kernel(x, edge_index, W1, W2, W3):
  N, F = x.shape
  E = edge_index.shape[1]
  H = W1.shape[1]
  C = W3.shape[1]
  f32 = jnp.float32

  ei = edge_index.astype(jnp.int32)

  def _tile_idx(v, ntiles, is_src):
    m = v.reshape(ntiles, -1)
    pad = _tile_rows(E, ntiles) * CH - m.shape[1]
    if is_src:
      padv = jnp.broadcast_to(
          (jnp.arange(pad, dtype=jnp.int32) * 797) % N, (ntiles, pad))
    else:
      padv = jnp.full((ntiles, pad), N, jnp.int32)
    return jnp.concatenate([m, padv], axis=1).reshape(ntiles, -1, CH)

  src16 = _tile_idx(ei[0], NS, True)
  dst16 = _tile_idx(ei[1], NS, False)
  src32 = _tile_idx(ei[0], NC * NS, True)
  dst32 = _tile_idx(ei[1], NC * NS, False)
  z64 = jnp.zeros((N, FW), f32)
  z16 = jnp.zeros((N, 16), f32)
  ones16 = jnp.ones((CH, 16), f32)
  W3p = jnp.pad(W3, ((0, 0), (0, FW - C)))

  BN = 2000
  grid = (N // BN,)

  degp = _make_deg(N, E)(dst32, ones16, z16)

  tc1 = pl.pallas_call(
      _tc1_body,
      grid=grid,
      in_specs=[
          pl.BlockSpec((BN, F), lambda i: (i, 0)),
          pl.BlockSpec((F, H), lambda i: (0, 0)),
          pl.BlockSpec((2, BN, 16), lambda i: (0, i, 0)),
      ],
      out_specs=pl.BlockSpec((2, BN, FW), lambda i: (0, i, 0)),
      out_shape=jax.ShapeDtypeStruct((2, N, FW), f32),
  )
  p1 = tc1(x, W1, degp)

  agg_split = _make_agg(N, E, True)
  s1 = agg_split(p1, src16, dst16, z64)

  tc2 = pl.pallas_call(
      functools.partial(_tcmid_body, True),
      grid=grid,
      in_specs=[
          pl.BlockSpec((2, BN, FW), lambda i: (0, i, 0)),
          pl.BlockSpec((2, BN, 16), lambda i: (0, i, 0)),
          pl.BlockSpec((H, H), lambda i: (0, 0)),
      ],
      out_specs=pl.BlockSpec((2, BN, FW), lambda i: (0, i, 0)),
      out_shape=jax.ShapeDtypeStruct((2, N, FW), f32),
  )
  p2 = tc2(s1, degp, W2)

  s2 = agg_split(p2, src16, dst16, z64)

  tc3 = pl.pallas_call(
      functools.partial(_tcmid_body, False),
      grid=grid,
      in_specs=[
          pl.BlockSpec((2, BN, FW), lambda i: (0, i, 0)),
          pl.BlockSpec((2, BN, 16), lambda i: (0, i, 0)),
          pl.BlockSpec((H, FW), lambda i: (0, 0)),
      ],
      out_specs=pl.BlockSpec((BN, FW), lambda i: (i, 0)),
      out_shape=jax.ShapeDtypeStruct((N, FW), f32),
  )
  p3 = tc3(s2, degp, W3p)

  s3 = _make_agg(N, E, False)(p3, src32, dst32, z64)

  tc4 = pl.pallas_call(
      functools.partial(_tc4_body, C),
      grid=grid,
      in_specs=[
          pl.BlockSpec((2, BN, FW), lambda i: (0, i, 0)),
          pl.BlockSpec((2, BN, 16), lambda i: (0, i, 0)),
      ],
      out_specs=pl.BlockSpec((BN, FW), lambda i: (i, 0)),
      out_shape=jax.ShapeDtypeStruct((N, FW), f32),
  )
  out = tc4(s3, degp)
  return out[:, :C]


# speedup vs baseline: 19.4346x; 1.1519x over previous
import functools

import jax
import jax.numpy as jnp
from jax import lax
from jax.experimental import pallas as pl
from jax.experimental.pallas import tpu as pltpu
from jax.experimental.pallas import tpu_sc as plsc

NC = 2
NS = 16
CH = 128
FW = 64
RS = 624
PAD = 8
QR = 40
K = 4


def _round_up(a, b):
  return (a + b - 1) // b * b


def _tile_rows(E, ntiles):
  return _round_up(-(-E // (ntiles * CH)), QR)


def _striped_rows(s, n, fn):
  r0 = pl.multiple_of(s * RS, 8)
  last = n - (NS - 1) * RS

  @pl.when(s < NS - 1)
  def _():
    fn(r0, RS)

  @pl.when(s == NS - 1)
  def _():
    fn(r0, last)


def _agg_body(split_feat, N, rpt, table_h, srca_h, srcb_h, dst_h, zero_h,
              out_h, acc_sp, idx_s, idx_d, rbufs, gsems, ssems):
  c = lax.axis_index("c")
  s = lax.axis_index("s")

  _striped_rows(s, N, lambda r0, nr: pltpu.sync_copy(
      zero_h.at[pl.ds(r0, nr), :], acc_sp.at[pl.ds(r0, nr), :]))

  plsc.subcore_barrier()

  w = s if split_feat else c * NS + s
  nq = rpt // QR
  nsup = QR // (2 * K)
  A = range(K)
  B = range(K, 2 * K)

  def start_g(j, b):
    pltpu.async_copy(table_h.at[idx_s.at[j]], rbufs[b], gsems[b])

  def wait_g(b):
    pltpu.make_async_copy(table_h.at[idx_s.at[0]], rbufs[b], gsems[b]).wait()

  def start_s(j, b):
    pltpu.async_copy(rbufs[b], acc_sp.at[idx_d.at[j]], ssems[b], add=True)

  def wait_s(b):
    pltpu.make_async_copy(rbufs[b], acc_sp.at[idx_d.at[0]], ssems[b]).wait()

  def superiter(u):
    j = 2 * K * u
    for k in A:
      wait_g(k)

    @pl.when(u > 0)
    def _():
      for k in B:
        wait_s(k)

    for k in B:
      start_g(j + k, k)
    for k in A:
      start_s(j + k, k)
    for k in B:
      wait_g(k)
    for k in A:
      wait_s(k)

    @pl.when(u < nsup - 1)
    def _():
      for k in A:
        start_g(j + 2 * K + k, k)
    for k in B:
      start_s(j + k, k)

  def quarter(q):
    q0 = pl.multiple_of(q * QR, 8)

    @pl.when(c == 0)
    def _():
      pltpu.sync_copy(srca_h.at[w, pl.ds(q0, QR), :], idx_s)

    @pl.when(c == 1)
    def _():
      pltpu.sync_copy(srcb_h.at[w, pl.ds(q0, QR), :], idx_s)
    pltpu.sync_copy(dst_h.at[w, pl.ds(q0, QR), :], idx_d)
    for k in A:
      start_g(k, k)
    lax.fori_loop(0, nsup, lambda u, _: (superiter(u), 0)[1], 0)
    for k in B:
      wait_s(k)

  lax.fori_loop(0, nq, lambda q, _: (quarter(q), 0)[1], 0)

  plsc.subcore_barrier()

  @pl.when(c == 0)
  def _():
    _striped_rows(s, N, lambda r0, nr: pltpu.sync_copy(
        acc_sp.at[pl.ds(r0, nr), :], out_h.at[pl.ds(r0, nr), pl.ds(0, FW)]))

  @pl.when(c == 1)
  def _():
    _striped_rows(s, N, lambda r0, nr: pltpu.sync_copy(
        acc_sp.at[pl.ds(r0, nr), :], out_h.at[pl.ds(r0, nr), pl.ds(FW, FW)]))


def _make_agg(N, E, split_feat):
  rpt = _tile_rows(E, NS if split_feat else NS * NC)
  mesh = plsc.VectorSubcoreMesh(core_axis_name="c", subcore_axis_name="s")
  return pl.kernel(
      functools.partial(_agg_body, split_feat, N, rpt),
      out_type=jax.ShapeDtypeStruct((N, 2 * FW), jnp.float32),
      mesh=mesh,
      scratch_types=[
          pltpu.VMEM_SHARED((N + PAD, FW), jnp.float32),
          pltpu.VMEM((QR, CH), jnp.int32),
          pltpu.VMEM((QR, CH), jnp.int32),
          [pltpu.VMEM((CH, FW), jnp.float32) for _ in range(2 * K)],
          [pltpu.SemaphoreType.DMA for _ in range(2 * K)],
          [pltpu.SemaphoreType.DMA for _ in range(2 * K)],
      ],
      name="gcn_agg",
      compiler_params=pltpu.CompilerParams(use_tc_tiling_on_sc=False),
  )


def _deg_body(N, rpt, dst_h, ones_h, zero_h, out_h, acc_sp, idx_d, ones_v, sem):
  c = lax.axis_index("c")
  s = lax.axis_index("s")
  _striped_rows(s, N, lambda r0, nr: pltpu.sync_copy(
      zero_h.at[pl.ds(r0, nr), :], acc_sp.at[pl.ds(r0, nr), :]))
  pltpu.sync_copy(ones_h, ones_v)
  pltpu.sync_copy(dst_h.at[c * NS + s], idx_d)
  plsc.subcore_barrier()

  def burst(b):
    lax.fori_loop(0, 20, lambda j, _: (
        pltpu.async_copy(ones_v, acc_sp.at[idx_d.at[b * 20 + j]], sem,
                         add=True), 0)[1], 0)
    lax.fori_loop(0, 20, lambda j, _: (
        pltpu.make_async_copy(ones_v, acc_sp.at[idx_d.at[0]], sem).wait(),
        0)[1], 0)

  lax.fori_loop(0, rpt // 20, lambda b, _: (burst(b), 0)[1], 0)
  plsc.subcore_barrier()

  @pl.when(c == 0)
  def _():
    _striped_rows(s, N, lambda r0, nr: pltpu.sync_copy(
        acc_sp.at[pl.ds(r0, nr), :], out_h.at[0, pl.ds(r0, nr), :]))

  @pl.when(c == 1)
  def _():
    _striped_rows(s, N, lambda r0, nr: pltpu.sync_copy(
        acc_sp.at[pl.ds(r0, nr), :], out_h.at[1, pl.ds(r0, nr), :]))


def _make_deg(N, E):
  rpt = _tile_rows(E, NS * NC)
  mesh = plsc.VectorSubcoreMesh(core_axis_name="c", subcore_axis_name="s")
  return pl.kernel(
      functools.partial(_deg_body, N, rpt),
      out_type=jax.ShapeDtypeStruct((2, N, 16), jnp.float32),
      mesh=mesh,
      scratch_types=[
          pltpu.VMEM_SHARED((N + PAD, 16), jnp.float32),
          pltpu.VMEM((_tile_rows(E, NS * NC), CH), jnp.int32),
          pltpu.VMEM((CH, 16), jnp.float32),
          pltpu.SemaphoreType.DMA,
      ],
      name="gcn_deg",
      compiler_params=pltpu.CompilerParams(use_tc_tiling_on_sc=False),
  )


def _dinv_of(degp_ref):
  deg = degp_ref[0, :, 0] + degp_ref[1, :, 0]
  return lax.rsqrt(jnp.maximum(deg, 1.0))


def _tc1_body(x_ref, w_ref, degp_ref, p_ref):
  dinv = _dinv_of(degp_ref)
  y = jnp.dot(x_ref[...], w_ref[...], preferred_element_type=jnp.float32)
  p_ref[...] = y * dinv[:, None]


def _tcmid_body(sh_ref, degp_ref, w_ref, p_ref):
  dinv = _dinv_of(degp_ref)
  h = jnp.maximum(sh_ref[...] * dinv[:, None], 0.0)
  y = jnp.dot(h, w_ref[...], preferred_element_type=jnp.float32)
  p_ref[...] = y * dinv[:, None]


def _tc4_body(ncls, sp_ref, degp_ref, out_ref):
  dinv = _dinv_of(degp_ref)
  sv = (sp_ref[:, :FW] + sp_ref[:, FW:]) * dinv[:, None]
  col = lax.broadcasted_iota(jnp.int32, sv.shape, 1)
  sv = jnp.where(col < ncls, sv, -1e30)
  m = jnp.max(sv, axis=1, keepdims=True)
  lse = jnp.log(jnp.sum(jnp.exp(sv - m), axis=1, keepdims=True)) + m
  out_ref[...] = sv - lse


def kernel(x, edge_index, W1, W2, W3):
  N, F = x.shape
  E = edge_index.shape[1]
  H = W1.shape[1]
  C = W3.shape[1]
  f32 = jnp.float32

  ei = edge_index.astype(jnp.int32)

  def _tile_idx(v, ntiles, pad_kind):
    m = v.reshape(ntiles, -1)
    pad = _tile_rows(E, ntiles) * CH - m.shape[1]
    if pad_kind == "src":
      padv = jnp.broadcast_to(
          ((jnp.arange(pad, dtype=jnp.int32) * 797) % N) * 2, (ntiles, pad))
    else:
      padv = jnp.full((ntiles, pad), N, jnp.int32)
    return jnp.concatenate([m, padv], axis=1).reshape(ntiles, -1, CH)

  srce = _tile_idx(ei[0] * 2, NC * NS, "src")
  srco = srce + 1
  dst32 = _tile_idx(ei[1], NC * NS, "dst")
  rpt16 = _tile_rows(E, NS)
  srce16 = srce.reshape(NS, rpt16, CH)
  srco16 = srco.reshape(NS, rpt16, CH)
  dst16 = dst32.reshape(NS, rpt16, CH)

  z64 = jnp.zeros((N, FW), f32)
  z16 = jnp.zeros((N, 16), f32)
  ones16 = jnp.ones((CH, 16), f32)
  W3p = jnp.pad(W3, ((0, 0), (0, 2 * FW - C)))

  BN = 2000
  grid = (N // BN,)

  degp = _make_deg(N, E)(dst32, ones16, z16)

  tc1 = pl.pallas_call(
      _tc1_body,
      grid=grid,
      in_specs=[
          pl.BlockSpec((BN, F), lambda i: (i, 0)),
          pl.BlockSpec((F, H), lambda i: (0, 0)),
          pl.BlockSpec((2, BN, 16), lambda i: (0, i, 0)),
      ],
      out_specs=pl.BlockSpec((BN, H), lambda i: (i, 0)),
      out_shape=jax.ShapeDtypeStruct((N, H), f32),
  )
  p1 = tc1(x, W1, degp)

  agg_split = _make_agg(N, E, True)
  s1 = agg_split(p1.reshape(2 * N, FW), srce16, srco16, dst16, z64)

  def tcmid(w_cols):
    return pl.pallas_call(
        _tcmid_body,
        grid=grid,
        in_specs=[
            pl.BlockSpec((BN, H), lambda i: (i, 0)),
            pl.BlockSpec((2, BN, 16), lambda i: (0, i, 0)),
            pl.BlockSpec((H, w_cols), lambda i: (0, 0)),
        ],
        out_specs=pl.BlockSpec((BN, w_cols), lambda i: (i, 0)),
        out_shape=jax.ShapeDtypeStruct((N, w_cols), f32),
    )

  p2 = tcmid(H)(s1, degp, W2)
  s2 = agg_split(p2.reshape(2 * N, FW), srce16, srco16, dst16, z64)

  p3 = tcmid(2 * FW)(s2, degp, W3p)
  s3 = _make_agg(N, E, False)(p3.reshape(2 * N, FW), srce, srce, dst32, z64)

  tc4 = pl.pallas_call(
      functools.partial(_tc4_body, C),
      grid=grid,
      in_specs=[
          pl.BlockSpec((BN, 2 * FW), lambda i: (i, 0)),
          pl.BlockSpec((2, BN, 16), lambda i: (0, i, 0)),
      ],
      out_specs=pl.BlockSpec((BN, FW), lambda i: (i, 0)),
      out_shape=jax.ShapeDtypeStruct((N, FW), f32),
  )
  out = tc4(s3, degp)
  return out[:, :C]


# speedup vs baseline: 19.8842x; 1.0231x over previous
import functools

import jax
import jax.numpy as jnp
from jax import lax
from jax.experimental import pallas as pl
from jax.experimental.pallas import tpu as pltpu
from jax.experimental.pallas import tpu_sc as plsc

NC = 2
NS = 16
CH = 128
FW = 64
RS = 624
PAD = 8
QR = 80
K = 4


def _round_up(a, b):
  return (a + b - 1) // b * b


def _tile_rows(E, ntiles):
  return _round_up(-(-E // (ntiles * CH)), QR)


def _striped_rows(s, n, fn):
  r0 = pl.multiple_of(s * RS, 8)
  last = n - (NS - 1) * RS

  @pl.when(s < NS - 1)
  def _():
    fn(r0, RS)

  @pl.when(s == NS - 1)
  def _():
    fn(r0, last)


def _agg_body(split_feat, N, rpt, table_h, srca_h, srcb_h, dst_h, zero_h,
              out_h, acc_sp, idx_s, idx_d, rbufs, gsems, ssems):
  c = lax.axis_index("c")
  s = lax.axis_index("s")

  _striped_rows(s, N, lambda r0, nr: pltpu.sync_copy(
      zero_h.at[pl.ds(r0, nr), :], acc_sp.at[pl.ds(r0, nr), :]))

  plsc.subcore_barrier()

  w = s if split_feat else c * NS + s
  nq = rpt // QR
  nsup = QR // (2 * K)
  A = range(K)
  B = range(K, 2 * K)

  def start_g(j, b):
    pltpu.async_copy(table_h.at[idx_s.at[j]], rbufs[b], gsems[b])

  def wait_g(b):
    pltpu.make_async_copy(table_h.at[idx_s.at[0]], rbufs[b], gsems[b]).wait()

  def start_s(j, b):
    pltpu.async_copy(rbufs[b], acc_sp.at[idx_d.at[j]], ssems[b], add=True)

  def wait_s(b):
    pltpu.make_async_copy(rbufs[b], acc_sp.at[idx_d.at[0]], ssems[b]).wait()

  def superiter(u):
    j = 2 * K * u
    for k in A:
      wait_g(k)

    @pl.when(u > 0)
    def _():
      for k in B:
        wait_s(k)

    for k in B:
      start_g(j + k, k)
    for k in A:
      start_s(j + k, k)
    for k in B:
      wait_g(k)
    for k in A:
      wait_s(k)

    @pl.when(u < nsup - 1)
    def _():
      for k in A:
        start_g(j + 2 * K + k, k)
    for k in B:
      start_s(j + k, k)

  def quarter(q):
    q0 = pl.multiple_of(q * QR, 8)

    @pl.when(c == 0)
    def _():
      pltpu.sync_copy(srca_h.at[w, pl.ds(q0, QR), :], idx_s)

    @pl.when(c == 1)
    def _():
      pltpu.sync_copy(srcb_h.at[w, pl.ds(q0, QR), :], idx_s)
    pltpu.sync_copy(dst_h.at[w, pl.ds(q0, QR), :], idx_d)
    for k in A:
      start_g(k, k)
    lax.fori_loop(0, nsup, lambda u, _: (superiter(u), 0)[1], 0)
    for k in B:
      wait_s(k)

  lax.fori_loop(0, nq, lambda q, _: (quarter(q), 0)[1], 0)

  plsc.subcore_barrier()

  @pl.when(c == 0)
  def _():
    _striped_rows(s, N, lambda r0, nr: pltpu.sync_copy(
        acc_sp.at[pl.ds(r0, nr), :], out_h.at[pl.ds(r0, nr), pl.ds(0, FW)]))

  @pl.when(c == 1)
  def _():
    _striped_rows(s, N, lambda r0, nr: pltpu.sync_copy(
        acc_sp.at[pl.ds(r0, nr), :], out_h.at[pl.ds(r0, nr), pl.ds(FW, FW)]))


def _make_agg(N, E, split_feat):
  rpt = _tile_rows(E, NS if split_feat else NS * NC)
  mesh = plsc.VectorSubcoreMesh(core_axis_name="c", subcore_axis_name="s")
  return pl.kernel(
      functools.partial(_agg_body, split_feat, N, rpt),
      out_type=jax.ShapeDtypeStruct((N, 2 * FW), jnp.float32),
      mesh=mesh,
      scratch_types=[
          pltpu.VMEM_SHARED((N + PAD, FW), jnp.float32),
          pltpu.VMEM((QR, CH), jnp.int32),
          pltpu.VMEM((QR, CH), jnp.int32),
          [pltpu.VMEM((CH, FW), jnp.float32) for _ in range(2 * K)],
          [pltpu.SemaphoreType.DMA for _ in range(2 * K)],
          [pltpu.SemaphoreType.DMA for _ in range(2 * K)],
      ],
      name="gcn_agg",
      compiler_params=pltpu.CompilerParams(use_tc_tiling_on_sc=False),
  )


def _deg_body(N, rpt, dst_h, ones_h, zero_h, out_h, acc_sp, idx_d, ones_v, sem):
  c = lax.axis_index("c")
  s = lax.axis_index("s")
  _striped_rows(s, N, lambda r0, nr: pltpu.sync_copy(
      zero_h.at[pl.ds(r0, nr), :], acc_sp.at[pl.ds(r0, nr), :]))
  pltpu.sync_copy(ones_h, ones_v)
  pltpu.sync_copy(dst_h.at[c * NS + s], idx_d)
  plsc.subcore_barrier()

  def burst(b):
    lax.fori_loop(0, 20, lambda j, _: (
        pltpu.async_copy(ones_v, acc_sp.at[idx_d.at[b * 20 + j]], sem,
                         add=True), 0)[1], 0)
    lax.fori_loop(0, 20, lambda j, _: (
        pltpu.make_async_copy(ones_v, acc_sp.at[idx_d.at[0]], sem).wait(),
        0)[1], 0)

  lax.fori_loop(0, rpt // 20, lambda b, _: (burst(b), 0)[1], 0)
  plsc.subcore_barrier()

  @pl.when(c == 0)
  def _():
    _striped_rows(s, N, lambda r0, nr: pltpu.sync_copy(
        acc_sp.at[pl.ds(r0, nr), :], out_h.at[0, pl.ds(r0, nr), :]))

  @pl.when(c == 1)
  def _():
    _striped_rows(s, N, lambda r0, nr: pltpu.sync_copy(
        acc_sp.at[pl.ds(r0, nr), :], out_h.at[1, pl.ds(r0, nr), :]))


def _make_deg(N, E):
  rpt = _tile_rows(E, NS * NC)
  mesh = plsc.VectorSubcoreMesh(core_axis_name="c", subcore_axis_name="s")
  return pl.kernel(
      functools.partial(_deg_body, N, rpt),
      out_type=jax.ShapeDtypeStruct((2, N, 16), jnp.float32),
      mesh=mesh,
      scratch_types=[
          pltpu.VMEM_SHARED((N + PAD, 16), jnp.float32),
          pltpu.VMEM((_tile_rows(E, NS * NC), CH), jnp.int32),
          pltpu.VMEM((CH, 16), jnp.float32),
          pltpu.SemaphoreType.DMA,
      ],
      name="gcn_deg",
      compiler_params=pltpu.CompilerParams(use_tc_tiling_on_sc=False),
  )


def _dinv_of(degp_ref):
  deg = degp_ref[0, :, 0] + degp_ref[1, :, 0]
  return lax.rsqrt(jnp.maximum(deg, 1.0))


def _tc1_body(x_ref, w_ref, degp_ref, p_ref):
  dinv = _dinv_of(degp_ref)
  y = jnp.dot(x_ref[...], w_ref[...], preferred_element_type=jnp.float32)
  p_ref[...] = y * dinv[:, None]


def _tcmid_body(sh_ref, degp_ref, w_ref, p_ref):
  dinv = _dinv_of(degp_ref)
  h = jnp.maximum(sh_ref[...] * dinv[:, None], 0.0)
  y = jnp.dot(h, w_ref[...], preferred_element_type=jnp.float32)
  p_ref[...] = y * dinv[:, None]


def _tc4_body(ncls, sp_ref, degp_ref, out_ref):
  dinv = _dinv_of(degp_ref)
  sv = (sp_ref[:, :FW] + sp_ref[:, FW:]) * dinv[:, None]
  col = lax.broadcasted_iota(jnp.int32, sv.shape, 1)
  sv = jnp.where(col < ncls, sv, -1e30)
  m = jnp.max(sv, axis=1, keepdims=True)
  lse = jnp.log(jnp.sum(jnp.exp(sv - m), axis=1, keepdims=True)) + m
  out_ref[...] = sv - lse


def kernel(x, edge_index, W1, W2, W3):
  N, F = x.shape
  E = edge_index.shape[1]
  H = W1.shape[1]
  C = W3.shape[1]
  f32 = jnp.float32

  ei = edge_index.astype(jnp.int32)

  def _tile_idx(v, ntiles, pad_kind):
    m = v.reshape(ntiles, -1)
    pad = _tile_rows(E, ntiles) * CH - m.shape[1]
    if pad_kind == "src":
      padv = jnp.broadcast_to(
          ((jnp.arange(pad, dtype=jnp.int32) * 797) % N) * 2, (ntiles, pad))
    else:
      padv = jnp.full((ntiles, pad), N, jnp.int32)
    return jnp.concatenate([m, padv], axis=1).reshape(ntiles, -1, CH)

  srce = _tile_idx(ei[0] * 2, NC * NS, "src")
  srco = srce + 1
  dst32 = _tile_idx(ei[1], NC * NS, "dst")
  rpt16 = _tile_rows(E, NS)
  srce16 = srce.reshape(NS, rpt16, CH)
  srco16 = srco.reshape(NS, rpt16, CH)
  dst16 = dst32.reshape(NS, rpt16, CH)

  z64 = jnp.zeros((N, FW), f32)
  z16 = jnp.zeros((N, 16), f32)
  ones16 = jnp.ones((CH, 16), f32)
  W3p = jnp.pad(W3, ((0, 0), (0, 2 * FW - C)))

  BN = 2000
  grid = (N // BN,)

  degp = _make_deg(N, E)(dst32, ones16, z16)

  tc1 = pl.pallas_call(
      _tc1_body,
      grid=grid,
      in_specs=[
          pl.BlockSpec((BN, F), lambda i: (i, 0)),
          pl.BlockSpec((F, H), lambda i: (0, 0)),
          pl.BlockSpec((2, BN, 16), lambda i: (0, i, 0)),
      ],
      out_specs=pl.BlockSpec((BN, H), lambda i: (i, 0)),
      out_shape=jax.ShapeDtypeStruct((N, H), f32),
  )
  p1 = tc1(x, W1, degp)

  agg_split = _make_agg(N, E, True)
  s1 = agg_split(p1.reshape(2 * N, FW), srce16, srco16, dst16, z64)

  def tcmid(w_cols):
    return pl.pallas_call(
        _tcmid_body,
        grid=grid,
        in_specs=[
            pl.BlockSpec((BN, H), lambda i: (i, 0)),
            pl.BlockSpec((2, BN, 16), lambda i: (0, i, 0)),
            pl.BlockSpec((H, w_cols), lambda i: (0, 0)),
        ],
        out_specs=pl.BlockSpec((BN, w_cols), lambda i: (i, 0)),
        out_shape=jax.ShapeDtypeStruct((N, w_cols), f32),
    )

  p2 = tcmid(H)(s1, degp, W2)
  s2 = agg_split(p2.reshape(2 * N, FW), srce16, srco16, dst16, z64)

  p3 = tcmid(2 * FW)(s2, degp, W3p)
  s3 = _make_agg(N, E, False)(p3.reshape(2 * N, FW), srce, srce, dst32, z64)

  tc4 = pl.pallas_call(
      functools.partial(_tc4_body, C),
      grid=grid,
      in_specs=[
          pl.BlockSpec((BN, 2 * FW), lambda i: (i, 0)),
          pl.BlockSpec((2, BN, 16), lambda i: (0, i, 0)),
      ],
      out_specs=pl.BlockSpec((BN, FW), lambda i: (i, 0)),
      out_shape=jax.ShapeDtypeStruct((N, FW), f32),
  )
  out = tc4(s3, degp)
  return out[:, :C]


# speedup vs baseline: 19.8950x; 1.0005x over previous
import functools

import jax
import jax.numpy as jnp
from jax import lax
from jax.experimental import pallas as pl
from jax.experimental.pallas import tpu as pltpu
from jax.experimental.pallas import tpu_sc as plsc

NC = 2
NS = 16
CH = 128
FW = 64
RS = 624
PAD = 8
QR = 80
K = 4


def _round_up(a, b):
  return (a + b - 1) // b * b


def _tile_rows(E, ntiles):
  return _round_up(-(-E // (ntiles * CH)), QR)


def _striped_rows(s, n, fn):
  r0 = pl.multiple_of(s * RS, 8)
  last = n - (NS - 1) * RS

  @pl.when(s < NS - 1)
  def _():
    fn(r0, RS)

  @pl.when(s == NS - 1)
  def _():
    fn(r0, last)


def _agg_body(split_feat, N, rpt, table_h, srca_h, srcb_h, dst_h, zero_h,
              out_h, acc_sp, idx_s, idx_d, rbufs, gsems, ssems):
  c = lax.axis_index("c")
  s = lax.axis_index("s")

  _striped_rows(s, N, lambda r0, nr: pltpu.sync_copy(
      zero_h.at[pl.ds(r0, nr), :], acc_sp.at[pl.ds(r0, nr), :]))

  plsc.subcore_barrier()

  w = s if split_feat else c * NS + s
  nq = rpt // QR
  nsup = QR // (2 * K)
  A = range(K)
  B = range(K, 2 * K)

  def start_g(j, b):
    pltpu.async_copy(table_h.at[idx_s.at[j]], rbufs[b], gsems[b])

  def wait_g(b):
    pltpu.make_async_copy(table_h.at[idx_s.at[0]], rbufs[b], gsems[b]).wait()

  def start_s(j, b):
    pltpu.async_copy(rbufs[b], acc_sp.at[idx_d.at[j]], ssems[b], add=True)

  def wait_s(b):
    pltpu.make_async_copy(rbufs[b], acc_sp.at[idx_d.at[0]], ssems[b]).wait()

  def superiter(u):
    j = 2 * K * u
    for k in A:
      wait_g(k)

    @pl.when(u > 0)
    def _():
      for k in B:
        wait_s(k)

    for k in B:
      start_g(j + k, k)
    for k in A:
      start_s(j + k, k)
    for k in B:
      wait_g(k)
    for k in A:
      wait_s(k)

    @pl.when(u < nsup - 1)
    def _():
      for k in A:
        start_g(j + 2 * K + k, k)
    for k in B:
      start_s(j + k, k)

  def quarter(q):
    q0 = pl.multiple_of(q * QR, 8)

    @pl.when(c == 0)
    def _():
      pltpu.sync_copy(srca_h.at[w, pl.ds(q0, QR), :], idx_s)

    @pl.when(c == 1)
    def _():
      pltpu.sync_copy(srcb_h.at[w, pl.ds(q0, QR), :], idx_s)
    pltpu.sync_copy(dst_h.at[w, pl.ds(q0, QR), :], idx_d)
    for k in A:
      start_g(k, k)
    lax.fori_loop(0, nsup, lambda u, _: (superiter(u), 0)[1], 0)
    for k in B:
      wait_s(k)

  lax.fori_loop(0, nq, lambda q, _: (quarter(q), 0)[1], 0)

  plsc.subcore_barrier()

  @pl.when(c == 0)
  def _():
    _striped_rows(s, N, lambda r0, nr: pltpu.sync_copy(
        acc_sp.at[pl.ds(r0, nr), :], out_h.at[pl.ds(r0, nr), pl.ds(0, FW)]))

  @pl.when(c == 1)
  def _():
    _striped_rows(s, N, lambda r0, nr: pltpu.sync_copy(
        acc_sp.at[pl.ds(r0, nr), :], out_h.at[pl.ds(r0, nr), pl.ds(FW, FW)]))


def _make_agg(N, E, split_feat):
  rpt = _tile_rows(E, NS if split_feat else NS * NC)
  mesh = plsc.VectorSubcoreMesh(core_axis_name="c", subcore_axis_name="s")
  return pl.kernel(
      functools.partial(_agg_body, split_feat, N, rpt),
      out_type=jax.ShapeDtypeStruct((N, 2 * FW), jnp.float32),
      mesh=mesh,
      scratch_types=[
          pltpu.VMEM_SHARED((N + PAD, FW), jnp.float32),
          pltpu.VMEM((QR, CH), jnp.int32),
          pltpu.VMEM((QR, CH), jnp.int32),
          [pltpu.VMEM((CH, FW), jnp.float32) for _ in range(2 * K)],
          [pltpu.SemaphoreType.DMA for _ in range(2 * K)],
          [pltpu.SemaphoreType.DMA for _ in range(2 * K)],
      ],
      name="gcn_agg",
      compiler_params=pltpu.CompilerParams(use_tc_tiling_on_sc=False),
  )


def _deg_body(N, rpt, dst_h, ones_h, zero_h, out_h, acc_sp, idx_d, ones_v, sem):
  c = lax.axis_index("c")
  s = lax.axis_index("s")
  _striped_rows(s, N, lambda r0, nr: pltpu.sync_copy(
      zero_h.at[pl.ds(r0, nr), :], acc_sp.at[pl.ds(r0, nr), :]))
  pltpu.sync_copy(ones_h, ones_v)
  pltpu.sync_copy(dst_h.at[c * NS + s], idx_d)
  plsc.subcore_barrier()

  def burst(b):
    lax.fori_loop(0, 20, lambda j, _: (
        pltpu.async_copy(ones_v, acc_sp.at[idx_d.at[b * 20 + j]], sem,
                         add=True), 0)[1], 0)
    lax.fori_loop(0, 20, lambda j, _: (
        pltpu.make_async_copy(ones_v, acc_sp.at[idx_d.at[0]], sem).wait(),
        0)[1], 0)

  lax.fori_loop(0, rpt // 20, lambda b, _: (burst(b), 0)[1], 0)
  plsc.subcore_barrier()

  @pl.when(c == 0)
  def _():
    _striped_rows(s, N, lambda r0, nr: pltpu.sync_copy(
        acc_sp.at[pl.ds(r0, nr), :], out_h.at[0, pl.ds(r0, nr), :]))

  @pl.when(c == 1)
  def _():
    _striped_rows(s, N, lambda r0, nr: pltpu.sync_copy(
        acc_sp.at[pl.ds(r0, nr), :], out_h.at[1, pl.ds(r0, nr), :]))


def _make_deg(N, E):
  rpt = _tile_rows(E, NS * NC)
  mesh = plsc.VectorSubcoreMesh(core_axis_name="c", subcore_axis_name="s")
  return pl.kernel(
      functools.partial(_deg_body, N, rpt),
      out_type=jax.ShapeDtypeStruct((2, N, 16), jnp.float32),
      mesh=mesh,
      scratch_types=[
          pltpu.VMEM_SHARED((N + PAD, 16), jnp.float32),
          pltpu.VMEM((_tile_rows(E, NS * NC), CH), jnp.int32),
          pltpu.VMEM((CH, 16), jnp.float32),
          pltpu.SemaphoreType.DMA,
      ],
      name="gcn_deg",
      compiler_params=pltpu.CompilerParams(use_tc_tiling_on_sc=False),
  )


def _dinv_of(degp_ref):
  deg = degp_ref[0, :, 0] + degp_ref[1, :, 0]
  return lax.rsqrt(jnp.maximum(deg, 1.0))


def _tc1_body(x_ref, w_ref, degp_ref, p_ref):
  dinv = _dinv_of(degp_ref)
  y = jnp.dot(x_ref[...], w_ref[...], preferred_element_type=jnp.float32)
  p_ref[...] = y * dinv[:, None]


def _tcmid_body(sh_ref, degp_ref, w_ref, p_ref):
  dinv = _dinv_of(degp_ref)
  h = jnp.maximum(sh_ref[...] * dinv[:, None], 0.0)
  y = jnp.dot(h, w_ref[...], preferred_element_type=jnp.float32)
  p_ref[...] = y * dinv[:, None]


def _tc4_body(ncls, sp_ref, degp_ref, out_ref):
  dinv = _dinv_of(degp_ref)
  sv = (sp_ref[:, :FW] + sp_ref[:, FW:]) * dinv[:, None]
  col = lax.broadcasted_iota(jnp.int32, sv.shape, 1)
  sv = jnp.where(col < ncls, sv, -1e30)
  m = jnp.max(sv, axis=1, keepdims=True)
  lse = jnp.log(jnp.sum(jnp.exp(sv - m), axis=1, keepdims=True)) + m
  out_ref[...] = (sv - lse)[:, :ncls]


def kernel(x, edge_index, W1, W2, W3):
  N, F = x.shape
  E = edge_index.shape[1]
  H = W1.shape[1]
  C = W3.shape[1]
  f32 = jnp.float32

  ei = edge_index.astype(jnp.int32)

  def _tile_idx(v, ntiles, pad_kind):
    m = v.reshape(ntiles, -1)
    pad = _tile_rows(E, ntiles) * CH - m.shape[1]
    if pad_kind == "src":
      padv = jnp.broadcast_to(
          ((jnp.arange(pad, dtype=jnp.int32) * 797) % N) * 2, (ntiles, pad))
    else:
      padv = jnp.full((ntiles, pad), N, jnp.int32)
    return jnp.concatenate([m, padv], axis=1).reshape(ntiles, -1, CH)

  srce = _tile_idx(ei[0] * 2, NC * NS, "src")
  srco = srce + 1
  dst32 = _tile_idx(ei[1], NC * NS, "dst")
  rpt16 = _tile_rows(E, NS)
  srce16 = srce.reshape(NS, rpt16, CH)
  srco16 = srco.reshape(NS, rpt16, CH)
  dst16 = dst32.reshape(NS, rpt16, CH)

  z64 = jnp.zeros((N, FW), f32)
  z16 = jnp.zeros((N, 16), f32)
  ones16 = jnp.ones((CH, 16), f32)
  W3p = jnp.pad(W3, ((0, 0), (0, 2 * FW - C)))

  BN = 2000
  grid = (N // BN,)

  degp = _make_deg(N, E)(dst32, ones16, z16)

  tc1 = pl.pallas_call(
      _tc1_body,
      grid=grid,
      in_specs=[
          pl.BlockSpec((BN, F), lambda i: (i, 0)),
          pl.BlockSpec((F, H), lambda i: (0, 0)),
          pl.BlockSpec((2, BN, 16), lambda i: (0, i, 0)),
      ],
      out_specs=pl.BlockSpec((BN, H), lambda i: (i, 0)),
      out_shape=jax.ShapeDtypeStruct((N, H), f32),
  )
  p1 = tc1(x, W1, degp)

  agg_split = _make_agg(N, E, True)
  s1 = agg_split(p1.reshape(2 * N, FW), srce16, srco16, dst16, z64)

  def tcmid(w_cols):
    return pl.pallas_call(
        _tcmid_body,
        grid=grid,
        in_specs=[
            pl.BlockSpec((BN, H), lambda i: (i, 0)),
            pl.BlockSpec((2, BN, 16), lambda i: (0, i, 0)),
            pl.BlockSpec((H, w_cols), lambda i: (0, 0)),
        ],
        out_specs=pl.BlockSpec((BN, w_cols), lambda i: (i, 0)),
        out_shape=jax.ShapeDtypeStruct((N, w_cols), f32),
    )

  p2 = tcmid(H)(s1, degp, W2)
  s2 = agg_split(p2.reshape(2 * N, FW), srce16, srco16, dst16, z64)

  p3 = tcmid(2 * FW)(s2, degp, W3p)
  s3 = _make_agg(N, E, False)(p3.reshape(2 * N, FW), srce, srce, dst32, z64)

  tc4 = pl.pallas_call(
      functools.partial(_tc4_body, C),
      grid=grid,
      in_specs=[
          pl.BlockSpec((BN, 2 * FW), lambda i: (i, 0)),
          pl.BlockSpec((2, BN, 16), lambda i: (0, i, 0)),
      ],
      out_specs=pl.BlockSpec((BN, C), lambda i: (i, 0)),
      out_shape=jax.ShapeDtypeStruct((N, C), f32),
  )
  return tc4(s3, degp)
